# Initial kernel scaffold; baseline (speedup 1.0000x reference)
import jax
import jax.numpy as jnp
from jax.experimental import pallas as pl


def kernel(position_ids, pos_enc):
    raise NotImplementedError("write your pallas kernel here")


# speedup vs baseline: 1.9816x; 1.9816x over previous
import functools

import jax
import jax.numpy as jnp
from jax import lax
from jax.experimental import pallas as pl
from jax.experimental.pallas import tpu as pltpu
from jax.experimental.pallas import tpu_sc as plsc

_HIDDEN = 1024
_NUM_CORES = 2
_NUM_SUBCORES = 16
_NUM_WORKERS = _NUM_CORES * _NUM_SUBCORES
_CHUNK = 32


def _gather_body(table_hbm, idx_hbm, out_hbm, idx_v, rows_v, sem):
    wid = lax.axis_index("s") * _NUM_CORES + lax.axis_index("c")
    n_total = idx_hbm.shape[0]
    b_per_w = n_total // _NUM_WORKERS
    base = wid * b_per_w
    pltpu.sync_copy(idx_hbm.at[pl.ds(base, b_per_w)], idx_v)

    def body(g, carry):
        off = g * _CHUNK
        pltpu.async_copy(
            table_hbm.at[idx_v.at[pl.ds(off, _CHUNK)]], rows_v, sem
        ).wait()
        pltpu.sync_copy(rows_v, out_hbm.at[pl.ds(base + off, _CHUNK)])
        return carry

    lax.fori_loop(0, b_per_w // _CHUNK, body, 0)


@functools.partial(jax.jit, static_argnums=(2,))
def _gather_call(table, idx, n_total):
    mesh = plsc.VectorSubcoreMesh(core_axis_name="c", subcore_axis_name="s")
    return pl.kernel(
        _gather_body,
        out_type=jax.ShapeDtypeStruct((n_total, _HIDDEN), jnp.float32),
        mesh=mesh,
        scratch_types=[
            pltpu.VMEM((n_total // _NUM_WORKERS,), jnp.int32),
            pltpu.VMEM((_CHUNK, _HIDDEN), jnp.float32),
            pltpu.SemaphoreType.DMA,
        ],
    )(table, idx)


def kernel(position_ids, pos_enc):
    b, s = position_ids.shape
    idx = position_ids.reshape(-1).astype(jnp.int32)
    out = _gather_call(pos_enc, idx, b * s)
    return out.reshape(b, s, pos_enc.shape[1])


# speedup vs baseline: 2.3808x; 1.2015x over previous
import functools

import jax
import jax.numpy as jnp
from jax import lax
from jax.experimental import pallas as pl
from jax.experimental.pallas import tpu as pltpu
from jax.experimental.pallas import tpu_sc as plsc

_HIDDEN = 1024
_NUM_CORES = 2
_NUM_SUBCORES = 16
_NUM_WORKERS = _NUM_CORES * _NUM_SUBCORES
_CHUNK = 32


def _gather_body(table_hbm, idx_hbm, out_hbm, idx_v, rows0, rows1, gsem0, gsem1):
    wid = lax.axis_index("s") * _NUM_CORES + lax.axis_index("c")
    n_total = idx_hbm.shape[0]
    b_per_w = n_total // _NUM_WORKERS
    base = wid * b_per_w
    n_chunks = b_per_w // _CHUNK
    pltpu.sync_copy(idx_hbm.at[pl.ds(base, b_per_w)], idx_v)

    def g_desc(c, rows, gsem):
        return pltpu.make_async_copy(
            table_hbm.at[idx_v.at[pl.ds(c * _CHUNK, _CHUNK)]], rows, gsem
        )

    g_desc(0, rows0, gsem0).start()
    g_desc(1, rows1, gsem1).start()

    def pair(i, carry):
        for b, rows, gsem in ((0, rows0, gsem0), (1, rows1, gsem1)):
            c = 2 * i + b
            g_desc(c, rows, gsem).wait()
            pltpu.sync_copy(rows, out_hbm.at[pl.ds(base + c * _CHUNK, _CHUNK)])

            @pl.when(c + 2 < n_chunks)
            def _():
                g_desc(c + 2, rows, gsem).start()

        return carry

    lax.fori_loop(0, n_chunks // 2, pair, 0)


@functools.partial(jax.jit, static_argnums=(2,))
def _gather_call(table, idx, n_total):
    mesh = plsc.VectorSubcoreMesh(core_axis_name="c", subcore_axis_name="s")
    return pl.kernel(
        _gather_body,
        out_type=jax.ShapeDtypeStruct((n_total, _HIDDEN), jnp.float32),
        mesh=mesh,
        scratch_types=[
            pltpu.VMEM((n_total // _NUM_WORKERS,), jnp.int32),
            pltpu.VMEM((_CHUNK, _HIDDEN), jnp.float32),
            pltpu.VMEM((_CHUNK, _HIDDEN), jnp.float32),
            pltpu.SemaphoreType.DMA,
            pltpu.SemaphoreType.DMA,
        ],
    )(table, idx)


def kernel(position_ids, pos_enc):
    b, s = position_ids.shape
    idx = position_ids.reshape(-1).astype(jnp.int32)
    out = _gather_call(pos_enc, idx, b * s)
    return out.reshape(b, s, pos_enc.shape[1])


# speedup vs baseline: 2.3876x; 1.0028x over previous
import functools

import jax
import jax.numpy as jnp
from jax import lax
from jax.experimental import pallas as pl
from jax.experimental.pallas import tpu as pltpu
from jax.experimental.pallas import tpu_sc as plsc

_HIDDEN = 1024
_NUM_CORES = 2
_NUM_SUBCORES = 16
_NUM_WORKERS = _NUM_CORES * _NUM_SUBCORES
_NBUF = 4
_CHUNK = 16


def _gather_body(table_hbm, idx_hbm, out_hbm, idx_v, rows, gsems, osems):
    wid = lax.axis_index("s") * _NUM_CORES + lax.axis_index("c")
    n_total = idx_hbm.shape[0]
    b_per_w = n_total // _NUM_WORKERS
    base = wid * b_per_w
    n_chunks = b_per_w // _CHUNK
    pltpu.sync_copy(idx_hbm.at[pl.ds(base, b_per_w)], idx_v)

    def g_desc(c, b):
        return pltpu.make_async_copy(
            table_hbm.at[idx_v.at[pl.ds(c * _CHUNK, _CHUNK)]], rows[b], gsems[b]
        )

    def o_desc(c, b):
        return pltpu.make_async_copy(
            rows[b], out_hbm.at[pl.ds(base + c * _CHUNK, _CHUNK)], osems[b]
        )

    g_desc(0, 0).start()
    g_desc(1, 1).start()

    def quad(i, carry):
        for b in range(_NBUF):
            c = _NBUF * i + b
            g_desc(c, b).wait()
            o_desc(c, b).start()
            nb = (b + 2) % _NBUF

            @pl.when(c + 2 < n_chunks)
            def _():
                @pl.when(c >= 2)
                def _():
                    o_desc(c - 2, nb).wait()

                g_desc(c + 2, nb).start()

        return carry

    lax.fori_loop(0, n_chunks // _NBUF, quad, 0)

    for k in range(_NBUF):
        c = n_chunks - _NBUF + k
        o_desc(c, c % _NBUF).wait()


@functools.partial(jax.jit, static_argnums=(2,))
def _gather_call(table, idx, n_total):
    mesh = plsc.VectorSubcoreMesh(core_axis_name="c", subcore_axis_name="s")
    return pl.kernel(
        _gather_body,
        out_type=jax.ShapeDtypeStruct((n_total, _HIDDEN), jnp.float32),
        mesh=mesh,
        scratch_types=[
            pltpu.VMEM((n_total // _NUM_WORKERS,), jnp.int32),
            [pltpu.VMEM((_CHUNK, _HIDDEN), jnp.float32) for _ in range(_NBUF)],
            [pltpu.SemaphoreType.DMA for _ in range(_NBUF)],
            [pltpu.SemaphoreType.DMA for _ in range(_NBUF)],
        ],
    )(table, idx)


def kernel(position_ids, pos_enc):
    b, s = position_ids.shape
    idx = position_ids.reshape(-1).astype(jnp.int32)
    out = _gather_call(pos_enc, idx, b * s)
    return out.reshape(b, s, pos_enc.shape[1])


# speedup vs baseline: 2.3904x; 1.0012x over previous
import functools

import jax
import jax.numpy as jnp
from jax import lax
from jax.experimental import pallas as pl
from jax.experimental.pallas import tpu as pltpu
from jax.experimental.pallas import tpu_sc as plsc

_NUM_CORES = 2
_NUM_SUBCORES = 16
_NUM_WORKERS = _NUM_CORES * _NUM_SUBCORES
_NBUF = 4
_CHUNK = 16


def _gather_body(table_hbm, idx_hbm, out_hbm, idx_v, rows, gsems, osems):
    wid = lax.axis_index("s") * _NUM_CORES + lax.axis_index("c")
    batch, seq = idx_hbm.shape
    b_per_w = (batch * seq) // _NUM_WORKERS
    w_per_b = seq // b_per_w
    bb = wid // w_per_b
    col = (wid % w_per_b) * b_per_w
    n_chunks = b_per_w // _CHUNK
    pltpu.sync_copy(idx_hbm.at[bb, pl.ds(col, b_per_w)], idx_v)

    def g_desc(c, b):
        return pltpu.make_async_copy(
            table_hbm.at[idx_v.at[pl.ds(c * _CHUNK, _CHUNK)]], rows[b], gsems[b]
        )

    def o_desc(c, b):
        return pltpu.make_async_copy(
            rows[b], out_hbm.at[bb, pl.ds(col + c * _CHUNK, _CHUNK)], osems[b]
        )

    g_desc(0, 0).start()
    g_desc(1, 1).start()

    def quad(i, carry):
        for b in range(_NBUF):
            c = _NBUF * i + b
            g_desc(c, b).wait()
            o_desc(c, b).start()
            nb = (b + 2) % _NBUF

            @pl.when(c + 2 < n_chunks)
            def _():
                @pl.when(c >= 2)
                def _():
                    o_desc(c - 2, nb).wait()

                g_desc(c + 2, nb).start()

        return carry

    lax.fori_loop(0, n_chunks // _NBUF, quad, 0)

    for k in range(_NBUF):
        c = n_chunks - _NBUF + k
        o_desc(c, c % _NBUF).wait()


@jax.jit
def _gather_call(table, idx):
    batch, seq = idx.shape
    mesh = plsc.VectorSubcoreMesh(core_axis_name="c", subcore_axis_name="s")
    return pl.kernel(
        _gather_body,
        out_type=jax.ShapeDtypeStruct((batch, seq, table.shape[1]), jnp.float32),
        mesh=mesh,
        scratch_types=[
            pltpu.VMEM(((batch * seq) // _NUM_WORKERS,), jnp.int32),
            [pltpu.VMEM((_CHUNK, table.shape[1]), jnp.float32) for _ in range(_NBUF)],
            [pltpu.SemaphoreType.DMA for _ in range(_NBUF)],
            [pltpu.SemaphoreType.DMA for _ in range(_NBUF)],
        ],
    )(table, idx)


def kernel(position_ids, pos_enc):
    return _gather_call(pos_enc, position_ids.astype(jnp.int32))


# speedup vs baseline: 2.3913x; 1.0004x over previous
import jax
import jax.numpy as jnp
from jax import lax
from jax.experimental import pallas as pl
from jax.experimental.pallas import tpu as pltpu
from jax.experimental.pallas import tpu_sc as plsc

_NUM_CORES = 2
_NUM_SUBCORES = 16
_NUM_WORKERS = _NUM_CORES * _NUM_SUBCORES
_NBUF = 4
_CHUNK = 16


def _gather_body(table_hbm, idx_hbm, out_hbm, idx_v, rows, gsems, osems):
    wid = lax.axis_index("s") * _NUM_CORES + lax.axis_index("c")
    batch, seq = idx_hbm.shape
    b_per_w = (batch * seq) // _NUM_WORKERS
    w_per_b = seq // b_per_w
    bb = wid // w_per_b
    col = (wid % w_per_b) * b_per_w
    n_chunks = b_per_w // _CHUNK
    pltpu.sync_copy(idx_hbm.at[bb, pl.ds(col, b_per_w)], idx_v)

    def g_desc(c, b):
        return pltpu.make_async_copy(
            table_hbm.at[idx_v.at[pl.ds(c * _CHUNK, _CHUNK)]], rows[b], gsems[b]
        )

    def o_desc(c, b):
        return pltpu.make_async_copy(
            rows[b], out_hbm.at[bb, pl.ds(col + c * _CHUNK, _CHUNK)], osems[b]
        )

    g_desc(0, 0).start()
    g_desc(1, 1).start()

    def quad(i, carry):
        for b in range(_NBUF):
            c = _NBUF * i + b
            g_desc(c, b).wait()
            o_desc(c, b).start()
            nb = (b + 2) % _NBUF

            @pl.when(c + 2 < n_chunks)
            def _():
                @pl.when(c >= 2)
                def _():
                    o_desc(c - 2, nb).wait()

                g_desc(c + 2, nb).start()

        return carry

    lax.fori_loop(0, n_chunks // _NBUF, quad, 0)

    for k in range(_NBUF):
        c = n_chunks - _NBUF + k
        o_desc(c, c % _NBUF).wait()


@jax.jit
def _gather_call(table, idx):
    batch, seq = idx.shape
    mesh = plsc.VectorSubcoreMesh(core_axis_name="c", subcore_axis_name="s")
    return pl.kernel(
        _gather_body,
        out_type=jax.ShapeDtypeStruct((batch, seq, table.shape[1]), jnp.float32),
        mesh=mesh,
        scratch_types=[
            pltpu.VMEM(((batch * seq) // _NUM_WORKERS,), jnp.int32),
            [pltpu.VMEM((_CHUNK, table.shape[1]), jnp.float32) for _ in range(_NBUF)],
            [pltpu.SemaphoreType.DMA for _ in range(_NBUF)],
            [pltpu.SemaphoreType.DMA for _ in range(_NBUF)],
        ],
    )(table, idx)


def kernel(position_ids, pos_enc):
    return _gather_call(pos_enc, position_ids.astype(jnp.int32))
